# baseline (speedup 1.0000x reference)
import jax
import jax.numpy as jnp
from jax.experimental import pallas as pl
from jax.experimental.pallas import tpu as pltpu
from jax.experimental.pallas import tpu_sc as plsc

_N = 100000
_D = 512
_G = 128
_EPS = 1e-07
_B = 4000
_NB = _N // _B

_NT = 68000
_NTB = _NT // _B

_NC = 2
_NS = 16
_NW = _NC * _NS
_CH = 16
_NCHUNK = (_N - _NT) // _CH
_NSL = _D // 16


def _sc_reduce_body(x_hbm, batch_hbm, sums_out, hist_out,
                    rowbuf, idxbuf, accum, histbuf, xsem, isem):
    c = jax.lax.axis_index("c")
    s = jax.lax.axis_index("s")
    wid = s * _NC + c

    zero16 = jnp.zeros((16,), jnp.float32)
    lane = jax.lax.broadcasted_iota(jnp.int32, (16,), 0)
    ones16 = jnp.ones((16,), jnp.float32)

    def _zacc(r, carry):
        for k in range(_NSL):
            accum[pl.ds(r * _D + 16 * k, 16)] = zero16
        return carry
    jax.lax.fori_loop(0, _G, _zacc, None)

    def _zh(r, carry):
        histbuf[pl.ds(16 * r, 16)] = zero16
        return carry
    jax.lax.fori_loop(0, _G, _zh, None)

    trips = jnp.where(wid < _NCHUNK % _NW,
                      _NCHUNK // _NW + 1, _NCHUNK // _NW)

    def _issue(i, slot):
        start = _NT + (wid + i * _NW) * _CH
        pltpu.async_copy(batch_hbm.at[pl.ds(start, _CH)],
                         idxbuf.at[slot], isem.at[slot])
        pltpu.async_copy(x_hbm.at[pl.ds(start, _CH)],
                         rowbuf.at[slot], xsem.at[slot])

    _issue(0, 0)

    def _chunk(i, carry):
        p = jax.lax.rem(i, 2)
        start = _NT + (wid + i * _NW) * _CH

        @pl.when(i + 1 < trips)
        def _prefetch():
            _issue(i + 1, 1 - p)

        pltpu.make_async_copy(batch_hbm.at[pl.ds(start, _CH)],
                              idxbuf.at[p], isem.at[p]).wait()
        pltpu.make_async_copy(x_hbm.at[pl.ds(start, _CH)],
                              rowbuf.at[p], xsem.at[p]).wait()

        b = idxbuf[p]
        plsc.addupdate_scatter(histbuf, [b * 16 + lane], ones16)

        gmin = jnp.min(b)
        gmax = jnp.max(b)

        @pl.when(gmin == gmax)
        def _uniform():
            base = jnp.full((16,), gmin * _D, jnp.int32) + lane
            for k in range(_NSL):
                v = rowbuf[p, 0, pl.ds(16 * k, 16)]
                acc = v * v
                for r in range(1, _CH):
                    v = rowbuf[p, r, pl.ds(16 * k, 16)]
                    acc = acc + v * v
                plsc.addupdate_scatter(accum, [base + 16 * k], acc)

        @pl.when(gmin != gmax)
        def _mixed():
            def _row(r, carry2):
                g = jnp.max(jnp.where(lane == r, b, 0))
                base = jnp.full((16,), g * _D, jnp.int32) + lane
                for k in range(_NSL):
                    v = rowbuf[p, r, pl.ds(16 * k, 16)]
                    plsc.addupdate_scatter(accum, [base + 16 * k], v * v)
                return carry2
            jax.lax.fori_loop(0, _CH, _row, None)

        return carry

    jax.lax.fori_loop(0, trips, _chunk, None)

    pltpu.sync_copy(accum, sums_out.at[wid])
    pltpu.sync_copy(histbuf, hist_out.at[wid])


def _onehot(batch_ref):
    b = batch_ref[0, 0, :]
    return (b[:, None] == jax.lax.broadcasted_iota(jnp.int32, (_B, _G), 1)
            ).astype(jnp.float32)


def _tc_reduce_body(batch_ref, x_ref, sums_ref, counts_ref):
    j = pl.program_id(0)
    onehot = _onehot(batch_ref)

    @pl.when(j == 0)
    def _init():
        sums_ref[...] = jnp.zeros_like(sums_ref)
        counts_ref[...] = jnp.zeros_like(counts_ref)

    x = x_ref[...]
    sums_ref[...] += jax.lax.dot_general(
        onehot, x * x, (((0,), (0,)), ((), ())),
        preferred_element_type=jnp.float32)
    counts_ref[...] += jnp.sum(onehot, axis=0)[:, None]


def _norm_body(batch_ref, x_ref, sums_tc_ref, counts_tc_ref,
               sums_sc_ref, hist_sc_ref, o_ref, scale_ref):
    j = pl.program_id(0)

    @pl.when(j == 0)
    def _scale():
        tot = sums_tc_ref[...] + jnp.sum(sums_sc_ref[...], axis=0)
        cnt = (counts_tc_ref[...][:, 0]
               + jnp.sum(jnp.sum(hist_sc_ref[...], axis=2), axis=0))
        cnt = jnp.maximum(cnt, 1.0)[:, None]
        scale_ref[...] = jax.lax.rsqrt(tot / cnt + _EPS)

    gathered = jnp.dot(_onehot(batch_ref), scale_ref[...],
                       preferred_element_type=jnp.float32)
    o_ref[...] = x_ref[...] * gathered


@jax.jit
def kernel(input, batch, num_graphs):
    del num_graphs
    batch32 = batch.astype(jnp.int32)
    batch3 = batch32.reshape(_NB, 1, _B)

    sc_reduce = pl.kernel(
        _sc_reduce_body,
        out_type=[
            jax.ShapeDtypeStruct((_NW, _G * _D), jnp.float32),
            jax.ShapeDtypeStruct((_NW, _G * 16), jnp.float32),
        ],
        mesh=plsc.VectorSubcoreMesh(
            core_axis_name="c", subcore_axis_name="s",
            num_cores=_NC, num_subcores=_NS),
        scratch_types=[
            pltpu.VMEM((2, _CH, _D), jnp.float32),
            pltpu.VMEM((2, _CH), jnp.int32),
            pltpu.VMEM((_G * _D,), jnp.float32),
            pltpu.VMEM((_G * 16,), jnp.float32),
            pltpu.SemaphoreType.DMA((2,)),
            pltpu.SemaphoreType.DMA((2,)),
        ],
        compiler_params=pltpu.CompilerParams(needs_layout_passes=False),
    )
    sums_sc, hist_sc = sc_reduce(input, batch32)

    sums_tc, counts_tc = pl.pallas_call(
        _tc_reduce_body,
        grid=(_NTB,),
        in_specs=[
            pl.BlockSpec((1, 1, _B), lambda j: (j, 0, 0)),
            pl.BlockSpec((_B, _D), lambda j: (j, 0)),
        ],
        out_specs=[
            pl.BlockSpec((_G, _D), lambda j: (0, 0)),
            pl.BlockSpec((_G, 1), lambda j: (0, 0)),
        ],
        out_shape=[
            jax.ShapeDtypeStruct((_G, _D), jnp.float32),
            jax.ShapeDtypeStruct((_G, 1), jnp.float32),
        ],
        compiler_params=pltpu.CompilerParams(
            dimension_semantics=("arbitrary",)),
    )(batch3, input)

    sums_sc = sums_sc.reshape(_NW, _G, _D)
    hist_sc = hist_sc.reshape(_NW, _G, 16)

    out = pl.pallas_call(
        _norm_body,
        grid=(_NB,),
        in_specs=[
            pl.BlockSpec((1, 1, _B), lambda j: (j, 0, 0)),
            pl.BlockSpec((_B, _D), lambda j: (j, 0)),
            pl.BlockSpec((_G, _D), lambda j: (0, 0)),
            pl.BlockSpec((_G, 1), lambda j: (0, 0)),
            pl.BlockSpec((_NW, _G, _D), lambda j: (0, 0, 0)),
            pl.BlockSpec((_NW, _G, 16), lambda j: (0, 0, 0)),
        ],
        out_specs=pl.BlockSpec((_B, _D), lambda j: (j, 0)),
        out_shape=jax.ShapeDtypeStruct((_N, _D), jnp.float32),
        scratch_shapes=[pltpu.VMEM((_G, _D), jnp.float32)],
        compiler_params=pltpu.CompilerParams(
            dimension_semantics=("arbitrary",)),
    )(batch3, input, sums_tc, counts_tc, sums_sc, hist_sc)
    return out


# speedup vs baseline: 1.0091x; 1.0091x over previous
import jax
import jax.numpy as jnp
from jax.experimental import pallas as pl
from jax.experimental.pallas import tpu as pltpu
from jax.experimental.pallas import tpu_sc as plsc

_N = 100000
_D = 512
_G = 128
_EPS = 1e-07
_B = 4000
_NB = _N // _B

_NT = 68000
_NTB = _NT // _B

_NC = 2
_NS = 16
_NW = _NC * _NS
_CH = 16
_NCHUNK = (_N - _NT) // _CH
_NSL = _D // 16


def _sc_reduce_body(x_hbm, batch_hbm, sums_out, hist_out,
                    rowbuf, idxbuf, accum, histbuf, xsem, isem):
    c = jax.lax.axis_index("c")
    s = jax.lax.axis_index("s")
    wid = s * _NC + c

    zero16 = jnp.zeros((16,), jnp.float32)
    lane = jax.lax.broadcasted_iota(jnp.int32, (16,), 0)
    ones16 = jnp.ones((16,), jnp.float32)

    def _zacc(r, carry):
        for k in range(_NSL):
            accum[pl.ds(r * _D + 16 * k, 16)] = zero16
        return carry
    jax.lax.fori_loop(0, _G, _zacc, None)

    def _zh(r, carry):
        histbuf[pl.ds(16 * r, 16)] = zero16
        return carry
    jax.lax.fori_loop(0, _G, _zh, None)

    trips = jnp.where(wid < _NCHUNK % _NW,
                      _NCHUNK // _NW + 1, _NCHUNK // _NW)

    def _issue(i, slot):
        start = _NT + (wid + i * _NW) * _CH
        pltpu.async_copy(batch_hbm.at[pl.ds(start, _CH)],
                         idxbuf.at[slot], isem.at[slot])
        pltpu.async_copy(x_hbm.at[pl.ds(start, _CH)],
                         rowbuf.at[slot], xsem.at[slot])

    _issue(0, 0)

    def _chunk(i, carry):
        p = jax.lax.rem(i, 2)
        start = _NT + (wid + i * _NW) * _CH

        @pl.when(i + 1 < trips)
        def _prefetch():
            _issue(i + 1, 1 - p)

        pltpu.make_async_copy(batch_hbm.at[pl.ds(start, _CH)],
                              idxbuf.at[p], isem.at[p]).wait()
        pltpu.make_async_copy(x_hbm.at[pl.ds(start, _CH)],
                              rowbuf.at[p], xsem.at[p]).wait()

        b = idxbuf[p]
        plsc.addupdate_scatter(histbuf, [b * 16 + lane], ones16)

        gmin = jnp.min(b)
        gmax = jnp.max(b)

        @pl.when(gmin == gmax)
        def _uniform():
            base = jnp.full((16,), gmin * _D, jnp.int32) + lane
            for k in range(_NSL):
                v = rowbuf[p, 0, pl.ds(16 * k, 16)]
                acc = v * v
                for r in range(1, _CH):
                    v = rowbuf[p, r, pl.ds(16 * k, 16)]
                    acc = acc + v * v
                plsc.addupdate_scatter(accum, [base + 16 * k], acc)

        @pl.when(gmin != gmax)
        def _mixed():
            def _row(r, carry2):
                g = jnp.max(jnp.where(lane == r, b, 0))
                base = jnp.full((16,), g * _D, jnp.int32) + lane
                for k in range(_NSL):
                    v = rowbuf[p, r, pl.ds(16 * k, 16)]
                    plsc.addupdate_scatter(accum, [base + 16 * k], v * v)
                return carry2
            jax.lax.fori_loop(0, _CH, _row, None)

        return carry

    jax.lax.fori_loop(0, trips, _chunk, None)

    pltpu.sync_copy(accum, sums_out.at[wid])
    pltpu.sync_copy(histbuf, hist_out.at[wid])


def _onehot(batch_ref):
    b = batch_ref[0, 0, :]
    return (b[:, None] == jax.lax.broadcasted_iota(jnp.int32, (_B, _G), 1)
            ).astype(jnp.float32)


def _tc_reduce_body(batch_ref, x_ref, sums_ref, counts_ref):
    j = pl.program_id(0)
    onehot = _onehot(batch_ref)

    @pl.when(j == 0)
    def _init():
        sums_ref[...] = jnp.zeros_like(sums_ref)
        counts_ref[...] = jnp.zeros_like(counts_ref)

    x = x_ref[...]
    sums_ref[...] += jax.lax.dot_general(
        onehot, x * x, (((0,), (0,)), ((), ())),
        preferred_element_type=jnp.float32)
    counts_ref[...] += jnp.sum(onehot, axis=0)[:, None]


def _norm_body(batch_ref, x_ref, sums_tc_ref, counts_tc_ref,
               sums_sc_ref, hist_sc_ref, o_ref, scale_ref):
    j = pl.program_id(0)

    @pl.when(j == 0)
    def _scale():
        tot = sums_tc_ref[...] + jnp.sum(sums_sc_ref[...], axis=0)
        cnt = (counts_tc_ref[...][:, 0]
               + jnp.sum(jnp.sum(hist_sc_ref[...], axis=2), axis=0))
        cnt = jnp.maximum(cnt, 1.0)[:, None]
        scale_ref[...] = jax.lax.rsqrt(tot / cnt + _EPS)

    gathered = jnp.dot(_onehot(batch_ref), scale_ref[...],
                       preferred_element_type=jnp.float32)
    o_ref[...] = x_ref[...] * gathered


@jax.jit
def kernel(input, batch, num_graphs):
    del num_graphs
    batch32 = batch.astype(jnp.int32)
    batch3 = batch32.reshape(_NB, 1, _B)

    sc_reduce = pl.kernel(
        _sc_reduce_body,
        out_type=[
            jax.ShapeDtypeStruct((_NW, _G * _D), jnp.float32),
            jax.ShapeDtypeStruct((_NW, _G * 16), jnp.float32),
        ],
        mesh=plsc.VectorSubcoreMesh(
            core_axis_name="c", subcore_axis_name="s",
            num_cores=_NC, num_subcores=_NS),
        scratch_types=[
            pltpu.VMEM((2, _CH, _D), jnp.float32),
            pltpu.VMEM((2, _CH), jnp.int32),
            pltpu.VMEM((_G * _D,), jnp.float32),
            pltpu.VMEM((_G * 16,), jnp.float32),
            pltpu.SemaphoreType.DMA((2,)),
            pltpu.SemaphoreType.DMA((2,)),
        ],
        compiler_params=pltpu.CompilerParams(needs_layout_passes=False,
                                             has_side_effects=False),
    )
    sums_sc, hist_sc = sc_reduce(input, batch32)

    sums_tc, counts_tc = pl.pallas_call(
        _tc_reduce_body,
        grid=(_NTB,),
        in_specs=[
            pl.BlockSpec((1, 1, _B), lambda j: (j, 0, 0)),
            pl.BlockSpec((_B, _D), lambda j: (j, 0)),
        ],
        out_specs=[
            pl.BlockSpec((_G, _D), lambda j: (0, 0)),
            pl.BlockSpec((_G, 1), lambda j: (0, 0)),
        ],
        out_shape=[
            jax.ShapeDtypeStruct((_G, _D), jnp.float32),
            jax.ShapeDtypeStruct((_G, 1), jnp.float32),
        ],
        compiler_params=pltpu.CompilerParams(
            dimension_semantics=("arbitrary",)),
        cost_estimate=pl.CostEstimate(
            flops=2 * _NT * _G * _D, transcendentals=0,
            bytes_accessed=4 * _NT * _D),
    )(batch3, input)

    sums_sc = sums_sc.reshape(_NW, _G, _D)
    hist_sc = hist_sc.reshape(_NW, _G, 16)

    out = pl.pallas_call(
        _norm_body,
        grid=(_NB,),
        in_specs=[
            pl.BlockSpec((1, 1, _B), lambda j: (j, 0, 0)),
            pl.BlockSpec((_B, _D), lambda j: (j, 0)),
            pl.BlockSpec((_G, _D), lambda j: (0, 0)),
            pl.BlockSpec((_G, 1), lambda j: (0, 0)),
            pl.BlockSpec((_NW, _G, _D), lambda j: (0, 0, 0)),
            pl.BlockSpec((_NW, _G, 16), lambda j: (0, 0, 0)),
        ],
        out_specs=pl.BlockSpec((_B, _D), lambda j: (j, 0)),
        out_shape=jax.ShapeDtypeStruct((_N, _D), jnp.float32),
        scratch_shapes=[pltpu.VMEM((_G, _D), jnp.float32)],
        compiler_params=pltpu.CompilerParams(
            dimension_semantics=("arbitrary",)),
    )(batch3, input, sums_tc, counts_tc, sums_sc, hist_sc)
    return out
